# baseline (speedup 1.0000x reference)
import functools

import jax
import jax.numpy as jnp
from jax.experimental import pallas as pl
from jax.experimental.pallas import tpu as pltpu

_B, _S, _D, _E, _TOPK = 4, 8192, 4096, 64, 8
_H = _D // 4
_N = _B * _S
_BLK_M = 512

_NEG_INF = float("-inf")


def _router_body(x_ref, w1_ref, b1_ref, w2_ref, b2_ref, t_ref,
                 gate_ref, rw_ref, se_ref):
    x = x_ref[...]
    h = jnp.dot(x, w1_ref[...], preferred_element_type=jnp.float32)
    h = jnp.maximum(h + b1_ref[...], 0.0)
    g = jnp.dot(h, w2_ref[...], preferred_element_type=jnp.float32)
    g = (g + b2_ref[...]) * (1.0 / t_ref[0])
    gate_ref[...] = g

    m = jnp.max(g, axis=-1, keepdims=True)
    e = jnp.exp(g - m)
    p = e / jnp.sum(e, axis=-1, keepdims=True)

    iota = jax.lax.broadcasted_iota(jnp.int32, p.shape, 1)
    vals = p
    top_v, top_i = [], []
    for _ in range(_TOPK):
        mx = jnp.max(vals, axis=-1, keepdims=True)
        hit = vals == mx
        ind = jnp.min(jnp.where(hit, iota, _E), axis=-1, keepdims=True)
        top_v.append(mx)
        top_i.append(ind)
        vals = jnp.where(iota == ind, _NEG_INF, vals)
    rw_ref[...] = jnp.concatenate(top_v, axis=-1)
    se_ref[...] = jnp.concatenate(top_i, axis=-1)


def _router_call_kwargs():
    return dict(
        grid=(_N // _BLK_M,),
        in_specs=[
            pl.BlockSpec((_BLK_M, _D), lambda i: (i, 0)),
            pl.BlockSpec((_D, _H), lambda i: (0, 0)),
            pl.BlockSpec((1, _H), lambda i: (0, 0)),
            pl.BlockSpec((_H, _E), lambda i: (0, 0)),
            pl.BlockSpec((1, _E), lambda i: (0, 0)),
            pl.BlockSpec(memory_space=pltpu.SMEM),
        ],
        out_specs=[
            pl.BlockSpec((_BLK_M, _E), lambda i: (i, 0)),
            pl.BlockSpec((_BLK_M, _TOPK), lambda i: (i, 0)),
            pl.BlockSpec((_BLK_M, _TOPK), lambda i: (i, 0)),
        ],
        out_shape=[
            jax.ShapeDtypeStruct((_N, _E), jnp.float32),
            jax.ShapeDtypeStruct((_N, _TOPK), jnp.float32),
            jax.ShapeDtypeStruct((_N, _TOPK), jnp.int32),
        ],
    )


@jax.jit
def kernel(hidden_states, W1, b1, W2, b2, temperature):
    x = hidden_states.reshape(_N, _D)
    gate, rw, se = pl.pallas_call(_router_body, **_router_call_kwargs())(
        x, W1, b1.reshape(1, _H), W2, b2.reshape(1, _E), temperature)
    return rw, se, gate


# speedup vs baseline: 1.2828x; 1.2828x over previous
import functools

import jax
import jax.numpy as jnp
from jax import lax
from jax.experimental import pallas as pl
from jax.experimental.pallas import tpu as pltpu
from jax.experimental.pallas import tpu_sc as plsc

_B, _S, _D, _E, _TOPK = 4, 8192, 4096, 64, 8
_H = _D // 4
_N = _B * _S
_BLK_M = 512

_NWORKERS = 32
_CHUNK = _N // _NWORKERS
_LANES = 16
_GROUPS = _CHUNK // _LANES

_NEG_INF = float("-inf")


def _tree(op, xs):
    xs = list(xs)
    while len(xs) > 1:
        nxt = [op(xs[i], xs[i + 1]) for i in range(0, len(xs) - 1, 2)]
        if len(xs) % 2:
            nxt.append(xs[-1])
        xs = nxt
    return xs[0]


def _mlp_body(x_ref, w1_ref, b1_ref, w2_ref, b2_ref, t_ref,
              gate_ref, gate_t_ref):
    x = x_ref[...]
    h = jnp.dot(x, w1_ref[...], preferred_element_type=jnp.float32)
    h = jnp.maximum(h + b1_ref[...], 0.0)
    g = jnp.dot(h, w2_ref[...], preferred_element_type=jnp.float32)
    g = (g + b2_ref[...]) * (1.0 / t_ref[0])
    gate_ref[...] = g
    gate_t_ref[...] = g.T


def _mlp_call_kwargs():
    return dict(
        grid=(_N // _BLK_M,),
        in_specs=[
            pl.BlockSpec((_BLK_M, _D), lambda i: (i, 0)),
            pl.BlockSpec((_D, _H), lambda i: (0, 0)),
            pl.BlockSpec((1, _H), lambda i: (0, 0)),
            pl.BlockSpec((_H, _E), lambda i: (0, 0)),
            pl.BlockSpec((1, _E), lambda i: (0, 0)),
            pl.BlockSpec(memory_space=pltpu.SMEM),
        ],
        out_specs=[
            pl.BlockSpec((_BLK_M, _E), lambda i: (i, 0)),
            pl.BlockSpec((_E, _BLK_M), lambda i: (0, i)),
        ],
        out_shape=[
            jax.ShapeDtypeStruct((_N, _E), jnp.float32),
            jax.ShapeDtypeStruct((_E, _N), jnp.float32),
        ],
    )


def _route_sc_body(gate_t_hbm, rw_t_hbm, se_t_hbm, gt_v, rw_v, se_v):
    wid = lax.axis_index("s") * 2 + lax.axis_index("c")
    base = wid * _CHUNK
    pltpu.sync_copy(gate_t_hbm.at[:, pl.ds(base, _CHUNK)], gt_v)

    def group(g, carry):
        offs = g * _LANES
        vals = [gt_v[e, pl.ds(offs, _LANES)] for e in range(_E)]
        m = _tree(jnp.maximum, vals)
        tv = [jnp.full((_LANES,), _NEG_INF, jnp.float32) for _ in range(_TOPK)]
        ti = [jnp.zeros((_LANES,), jnp.int32) for _ in range(_TOPK)]
        ex = [jnp.exp(v - m) for v in vals]
        s = _tree(jnp.add, ex)
        for e in range(_E):
            v = ex[e]
            iv = jnp.full((_LANES,), e, jnp.int32)
            for j in range(_TOPK):
                gt = v > tv[j]
                nv = jnp.where(gt, v, tv[j])
                ni = jnp.where(gt, iv, ti[j])
                v = jnp.where(gt, tv[j], v)
                iv = jnp.where(gt, ti[j], iv)
                tv[j] = nv
                ti[j] = ni
        r = 1.0 / s
        for j in range(_TOPK):
            rw_v[j, pl.ds(offs, _LANES)] = tv[j] * r
            se_v[j, pl.ds(offs, _LANES)] = ti[j]
        return carry

    lax.fori_loop(0, _GROUPS, group, 0)
    pltpu.sync_copy(rw_v, rw_t_hbm.at[:, pl.ds(base, _CHUNK)])
    pltpu.sync_copy(se_v, se_t_hbm.at[:, pl.ds(base, _CHUNK)])


def _route_sc():
    mesh = plsc.VectorSubcoreMesh(core_axis_name="c", subcore_axis_name="s",
                                  num_cores=2, num_subcores=16)
    return pl.kernel(
        _route_sc_body,
        out_type=[
            jax.ShapeDtypeStruct((_TOPK, _N), jnp.float32),
            jax.ShapeDtypeStruct((_TOPK, _N), jnp.int32),
        ],
        mesh=mesh,
        scratch_types=[
            pltpu.VMEM((_E, _CHUNK), jnp.float32),
            pltpu.VMEM((_TOPK, _CHUNK), jnp.float32),
            pltpu.VMEM((_TOPK, _CHUNK), jnp.int32),
        ],
    )


@jax.jit
def kernel(hidden_states, W1, b1, W2, b2, temperature):
    x = hidden_states.reshape(_N, _D)
    gate, gate_t = pl.pallas_call(_mlp_body, **_mlp_call_kwargs())(
        x, W1, b1.reshape(1, _H), W2, b2.reshape(1, _E), temperature)
    rw_t, se_t = _route_sc()(gate_t)
    return rw_t.T, se_t.T, gate


# speedup vs baseline: 1.3311x; 1.0377x over previous
import functools

import jax
import jax.numpy as jnp
from jax import lax
from jax.experimental import pallas as pl
from jax.experimental.pallas import tpu as pltpu
from jax.experimental.pallas import tpu_sc as plsc

_B, _S, _D, _E, _TOPK = 4, 8192, 4096, 64, 8
_H = _D // 4
_N = _B * _S
_BLK_M = 1024

_NWORKERS = 32
_CHUNK = _N // _NWORKERS
_LANES = 16
_GROUPS = _CHUNK // _LANES

_NEG_INF = float("-inf")


def _tree(op, xs):
    xs = list(xs)
    while len(xs) > 1:
        nxt = [op(xs[i], xs[i + 1]) for i in range(0, len(xs) - 1, 2)]
        if len(xs) % 2:
            nxt.append(xs[-1])
        xs = nxt
    return xs[0]


def _mlp_body(x_ref, w1_ref, b1_ref, w2_ref, b2_ref, t_ref,
              gate_ref, gate_t_ref):
    x = x_ref[...]
    h = jnp.dot(x, w1_ref[...], preferred_element_type=jnp.float32)
    h = jnp.maximum(h + b1_ref[...], 0.0)
    g = jnp.dot(h, w2_ref[...], preferred_element_type=jnp.float32)
    g = (g + b2_ref[...]) * (1.0 / t_ref[0])
    gate_ref[...] = g
    gate_t_ref[...] = g.T


def _mlp_call_kwargs():
    return dict(
        grid=(_N // _BLK_M,),
        in_specs=[
            pl.BlockSpec((_BLK_M, _D), lambda i: (i, 0)),
            pl.BlockSpec((_D, _H), lambda i: (0, 0)),
            pl.BlockSpec((1, _H), lambda i: (0, 0)),
            pl.BlockSpec((_H, _E), lambda i: (0, 0)),
            pl.BlockSpec((1, _E), lambda i: (0, 0)),
            pl.BlockSpec(memory_space=pltpu.SMEM),
        ],
        out_specs=[
            pl.BlockSpec((_BLK_M, _E), lambda i: (i, 0)),
            pl.BlockSpec((_E, _BLK_M), lambda i: (0, i)),
        ],
        out_shape=[
            jax.ShapeDtypeStruct((_N, _E), jnp.float32),
            jax.ShapeDtypeStruct((_E, _N), jnp.float32),
        ],
    )


def _route_sc_body(gate_t_hbm, rw_t_hbm, se_t_hbm, gt_v, rw_v, se_v):
    wid = lax.axis_index("s") * 2 + lax.axis_index("c")
    base = wid * _CHUNK
    pltpu.sync_copy(gate_t_hbm.at[:, pl.ds(base, _CHUNK)], gt_v)

    def group(g, carry):
        offs = g * _LANES
        vals = [gt_v[e, pl.ds(offs, _LANES)] for e in range(_E)]
        m = _tree(jnp.maximum, vals)
        tv = [jnp.full((_LANES,), _NEG_INF, jnp.float32) for _ in range(_TOPK)]
        ti = [jnp.zeros((_LANES,), jnp.int32) for _ in range(_TOPK)]
        ex = [jnp.exp(v - m) for v in vals]
        s = _tree(jnp.add, ex)
        for e in range(_E):
            v = ex[e]
            iv = jnp.full((_LANES,), e, jnp.int32)
            for j in range(_TOPK):
                gt = v > tv[j]
                nv = jnp.where(gt, v, tv[j])
                ni = jnp.where(gt, iv, ti[j])
                v = jnp.where(gt, tv[j], v)
                iv = jnp.where(gt, ti[j], iv)
                tv[j] = nv
                ti[j] = ni
        r = 1.0 / s
        for j in range(_TOPK):
            rw_v[j, pl.ds(offs, _LANES)] = tv[j] * r
            se_v[j, pl.ds(offs, _LANES)] = ti[j]
        return carry

    lax.fori_loop(0, _GROUPS, group, 0)
    pltpu.sync_copy(rw_v, rw_t_hbm.at[:, pl.ds(base, _CHUNK)])
    pltpu.sync_copy(se_v, se_t_hbm.at[:, pl.ds(base, _CHUNK)])


def _route_sc():
    mesh = plsc.VectorSubcoreMesh(core_axis_name="c", subcore_axis_name="s",
                                  num_cores=2, num_subcores=16)
    return pl.kernel(
        _route_sc_body,
        out_type=[
            jax.ShapeDtypeStruct((_TOPK, _N), jnp.float32),
            jax.ShapeDtypeStruct((_TOPK, _N), jnp.int32),
        ],
        mesh=mesh,
        scratch_types=[
            pltpu.VMEM((_E, _CHUNK), jnp.float32),
            pltpu.VMEM((_TOPK, _CHUNK), jnp.float32),
            pltpu.VMEM((_TOPK, _CHUNK), jnp.int32),
        ],
    )


@jax.jit
def kernel(hidden_states, W1, b1, W2, b2, temperature):
    x = hidden_states.reshape(_N, _D)
    gate, gate_t = pl.pallas_call(_mlp_body, **_mlp_call_kwargs())(
        x, W1, b1.reshape(1, _H), W2, b2.reshape(1, _E), temperature)
    rw_t, se_t = _route_sc()(gate_t)
    return rw_t.T, se_t.T, gate
